# Initial kernel scaffold; baseline (speedup 1.0000x reference)
import jax
import jax.numpy as jnp
from jax.experimental import pallas as pl


def kernel(logits):
    raise NotImplementedError("write your pallas kernel here")


# speedup vs baseline: 75.3749x; 75.3749x over previous
import functools

import numpy as np
import jax
import jax.numpy as jnp
from jax import lax
from jax.experimental import pallas as pl
from jax.experimental.pallas import tpu as pltpu
from jax.experimental.pallas import tpu_sc as plsc

_Y = 1024
_X = 4094
_NVEC = (512 + (np.arange(_Y) * 37) % 3584).astype(np.int64)
_TOTAL = int(_NVEC.sum())

_CK = tuple(int(-(-3584 * k // 37)) for k in range(1, 11))

_NW = 32
_ROWS_PER_W = _Y // _NW
_XPAD = 4096
_INBUF = 4112
_PAD = _INBUF + 16

_mesh = plsc.VectorSubcoreMesh(core_axis_name="c", subcore_axis_name="s")


@functools.partial(
    pl.kernel,
    mesh=_mesh,
    out_type=jax.ShapeDtypeStruct((_Y, _XPAD), jnp.float32),
    scratch_types=[
        pltpu.VMEM((_INBUF,), jnp.float32),
        pltpu.VMEM((_XPAD,), jnp.float32),
    ],
)
def _sc_scatter(logits_hbm, grid_hbm, inbuf, outbuf):
    wid = lax.axis_index("s") * 2 + lax.axis_index("c")
    lane = lax.iota(jnp.int32, 16)
    neg_inf = jnp.full((16,), -jnp.inf, dtype=jnp.float32)

    def do_row(j, carry):
        r = wid * _ROWS_PER_W + j
        t = 37 * r
        nv = 512 + lax.rem(t, 3584)
        tri = (t * (r - 1)) // 2
        tcount = 0
        for ck in _CK:
            tcount = tcount + lax.max(0, r - ck)
        off = 512 * r + tri - 3584 * tcount
        sh = lax.rem(off, 8)
        a = pl.multiple_of(off - sh, 8)
        pltpu.sync_copy(logits_hbm.at[pl.ds(a, _INBUF)], inbuf)

        def do_vec(c, carry2):
            v = inbuf[pl.ds(sh + 16 * c, 16)]
            col = 16 * c + lane
            outbuf[pl.ds(16 * c, 16)] = jnp.where(col < nv, v, neg_inf)
            return carry2

        lax.fori_loop(0, _XPAD // 16, do_vec, 0)
        pltpu.sync_copy(outbuf, grid_hbm.at[r])
        return carry

    lax.fori_loop(0, _ROWS_PER_W, do_row, 0)


def _lsm_body(gw_ref, g_ref, lp_ref):
    g = gw_ref[...]
    m = jnp.max(g, axis=-1, keepdims=True)
    e = jnp.exp(g - m)
    s = jnp.sum(e, axis=-1, keepdims=True)
    lp = g - (m + jnp.log(s))
    g_ref[...] = g[:, :_X]
    lp_ref[...] = lp[:, :_X]


_BR = 128


def _log_softmax(grid_wide):
    return pl.pallas_call(
        _lsm_body,
        grid=(_Y // _BR,),
        in_specs=[pl.BlockSpec((_BR, _XPAD), lambda i: (i, 0))],
        out_specs=[
            pl.BlockSpec((_BR, _X), lambda i: (i, 0)),
            pl.BlockSpec((_BR, _X), lambda i: (i, 0)),
        ],
        out_shape=[
            jax.ShapeDtypeStruct((_Y, _X), jnp.float32),
            jax.ShapeDtypeStruct((_Y, _X), jnp.float32),
        ],
    )(grid_wide)


def kernel(logits):
    logits_pad = jnp.concatenate(
        [logits, jnp.zeros((_PAD,), jnp.float32)])
    grid_wide = _sc_scatter(logits_pad)
    grid, log_probs = _log_softmax(grid_wide)
    return (grid, log_probs)


# speedup vs baseline: 106.1996x; 1.4090x over previous
import functools

import numpy as np
import jax
import jax.numpy as jnp
from jax import lax
from jax.experimental import pallas as pl
from jax.experimental.pallas import tpu as pltpu
from jax.experimental.pallas import tpu_sc as plsc

_Y = 1024
_X = 4094
_NVEC = (512 + (np.arange(_Y) * 37) % 3584).astype(np.int64)
_TOTAL = int(_NVEC.sum())

_CK = tuple(int(-(-3584 * k // 37)) for k in range(1, 11))

_NW = 32
_ROWS_PER_W = _Y // _NW
_XPAD = 4096
_INBUF = 4112
_PAD = _INBUF + 16

_mesh = plsc.VectorSubcoreMesh(core_axis_name="c", subcore_axis_name="s")


def _row_params(r):
    t = 37 * r
    nv = 512 + lax.rem(t, 3584)
    tri = (t * (r - 1)) // 2
    tcount = 0
    for ck in _CK:
        tcount = tcount + lax.max(0, r - ck)
    off = 512 * r + tri - 3584 * tcount
    sh = lax.rem(off, 8)
    a = pl.multiple_of(off - sh, 8)
    return a, sh, nv


@functools.partial(
    pl.kernel,
    mesh=_mesh,
    out_type=jax.ShapeDtypeStruct((_Y, _XPAD), jnp.float32),
    scratch_types=[
        pltpu.VMEM((_INBUF,), jnp.float32),
        pltpu.VMEM((_INBUF,), jnp.float32),
        pltpu.VMEM((_XPAD,), jnp.float32),
        pltpu.VMEM((_XPAD,), jnp.float32),
        pltpu.SemaphoreType.DMA,
        pltpu.SemaphoreType.DMA,
        pltpu.SemaphoreType.DMA,
        pltpu.SemaphoreType.DMA,
    ],
)
def _sc_scatter(logits_hbm, grid_hbm, inbuf0, inbuf1, outbuf0, outbuf1,
                si0, si1, so0, so1):
    wid = lax.axis_index("s") * 2 + lax.axis_index("c")
    lane = lax.iota(jnp.int32, 16)
    neg_inf = jnp.full((16,), -jnp.inf, dtype=jnp.float32)
    inbufs, outbufs = (inbuf0, inbuf1), (outbuf0, outbuf1)
    sis, sos = (si0, si1), (so0, so1)
    r0 = wid * _ROWS_PER_W

    params = [_row_params(r0 + j) for j in range(_ROWS_PER_W)]

    for j in range(2):
        a, _, _ = params[j]
        pltpu.async_copy(logits_hbm.at[pl.ds(a, _INBUF)], inbufs[j], sis[j])

    prev16 = [_XPAD // 16, _XPAD // 16]

    for j in range(_ROWS_PER_W):
        b = j % 2
        inbuf, outbuf = inbufs[b], outbufs[b]
        a, sh, nv = params[j]
        nv16 = (nv + 15) // 16
        pltpu.make_async_copy(
            logits_hbm.at[pl.ds(0, _INBUF)], inbuf, sis[b]).wait()
        if j >= 2:
            pltpu.make_async_copy(
                logits_hbm.at[pl.ds(0, _XPAD)], outbuf, sos[b]).wait()

        def copy_vec(c, carry, inbuf=inbuf, outbuf=outbuf, sh=sh):
            outbuf[pl.ds(16 * c, 16)] = inbuf[pl.ds(sh + 16 * c, 16)]
            return carry

        lax.fori_loop(0, nv16 - 1, copy_vec, 0)
        cb = nv16 - 1
        v = inbuf[pl.ds(sh + 16 * cb, 16)]
        outbuf[pl.ds(16 * cb, 16)] = jnp.where(16 * cb + lane < nv, v, neg_inf)

        def poison_vec(c, carry, outbuf=outbuf):
            outbuf[pl.ds(16 * c, 16)] = neg_inf
            return carry

        lax.fori_loop(nv16, prev16[b], poison_vec, 0)
        prev16[b] = nv16

        pltpu.async_copy(outbuf, grid_hbm.at[r0 + j], sos[b])
        if j + 2 < _ROWS_PER_W:
            a2, _, _ = params[j + 2]
            pltpu.async_copy(
                logits_hbm.at[pl.ds(a2, _INBUF)], inbuf, sis[b])

    for b in range(2):
        pltpu.make_async_copy(
            logits_hbm.at[pl.ds(0, _XPAD)], outbufs[b], sos[b]).wait()


def _lsm_body(gw_ref, g_ref, lp_ref):
    g = gw_ref[...]
    m = jnp.max(g, axis=-1, keepdims=True)
    e = jnp.exp(g - m)
    s = jnp.sum(e, axis=-1, keepdims=True)
    lp = g - (m + jnp.log(s))
    g_ref[...] = g[:, :_X]
    lp_ref[...] = lp[:, :_X]


_BR = 128


def _log_softmax(grid_wide):
    return pl.pallas_call(
        _lsm_body,
        grid=(_Y // _BR,),
        in_specs=[pl.BlockSpec((_BR, _XPAD), lambda i: (i, 0))],
        out_specs=[
            pl.BlockSpec((_BR, _X), lambda i: (i, 0)),
            pl.BlockSpec((_BR, _X), lambda i: (i, 0)),
        ],
        out_shape=[
            jax.ShapeDtypeStruct((_Y, _X), jnp.float32),
            jax.ShapeDtypeStruct((_Y, _X), jnp.float32),
        ],
    )(grid_wide)


def kernel(logits):
    logits_pad = jnp.concatenate(
        [logits, jnp.zeros((_PAD,), jnp.float32)])
    grid_wide = _sc_scatter(logits_pad)
    grid, log_probs = _log_softmax(grid_wide)
    return (grid, log_probs)


# speedup vs baseline: 108.2211x; 1.0190x over previous
import functools

import numpy as np
import jax
import jax.numpy as jnp
from jax import lax
from jax.experimental import pallas as pl
from jax.experimental.pallas import tpu as pltpu
from jax.experimental.pallas import tpu_sc as plsc

_Y = 1024
_X = 4094
_NVEC = (512 + (np.arange(_Y) * 37) % 3584).astype(np.int64)
_TOTAL = int(_NVEC.sum())

_CK = tuple(int(-(-3584 * k // 37)) for k in range(1, 11))

_NW = 32
_ROWS_PER_W = _Y // _NW
_XPAD = 4096
_INBUF = 4112
_PAD = _INBUF + 16

_mesh = plsc.VectorSubcoreMesh(core_axis_name="c", subcore_axis_name="s")


def _row_params(r):
    t = 37 * r
    nv = 512 + lax.rem(t, 3584)
    tri = (t * (r - 1)) // 2
    tcount = 0
    for ck in _CK:
        tcount = tcount + lax.max(0, r - ck)
    off = 512 * r + tri - 3584 * tcount
    sh = lax.rem(off, 8)
    a = pl.multiple_of(off - sh, 8)
    return a, sh, nv


@functools.partial(
    pl.kernel,
    mesh=_mesh,
    out_type=jax.ShapeDtypeStruct((_Y, _XPAD), jnp.float32),
    scratch_types=[
        pltpu.VMEM((_INBUF,), jnp.float32),
        pltpu.VMEM((_INBUF,), jnp.float32),
        pltpu.VMEM((_XPAD,), jnp.float32),
        pltpu.VMEM((_XPAD,), jnp.float32),
        pltpu.SemaphoreType.DMA,
        pltpu.SemaphoreType.DMA,
        pltpu.SemaphoreType.DMA,
        pltpu.SemaphoreType.DMA,
    ],
)
def _sc_scatter(logits_hbm, grid_hbm, inbuf0, inbuf1, outbuf0, outbuf1,
                si0, si1, so0, so1):
    wid = lax.axis_index("s") * 2 + lax.axis_index("c")
    lane = lax.iota(jnp.int32, 16)
    neg_inf = jnp.full((16,), -jnp.inf, dtype=jnp.float32)
    inbufs, outbufs = (inbuf0, inbuf1), (outbuf0, outbuf1)
    sis, sos = (si0, si1), (so0, so1)
    r0 = wid * _ROWS_PER_W

    params = [_row_params(r0 + j) for j in range(_ROWS_PER_W)]

    for j in range(2):
        a, _, _ = params[j]
        pltpu.async_copy(logits_hbm.at[pl.ds(a, _INBUF)], inbufs[j], sis[j])

    prev16 = [_XPAD // 16, _XPAD // 16]

    for j in range(_ROWS_PER_W):
        b = j % 2
        inbuf, outbuf = inbufs[b], outbufs[b]
        a, sh, nv = params[j]
        nv16 = (nv + 15) // 16
        pltpu.make_async_copy(
            logits_hbm.at[pl.ds(0, _INBUF)], inbuf, sis[b]).wait()
        if j >= 2:
            pltpu.make_async_copy(
                logits_hbm.at[pl.ds(0, _XPAD)], outbuf, sos[b]).wait()

        def copy_vec(c, carry, inbuf=inbuf, outbuf=outbuf, sh=sh):
            outbuf[pl.ds(16 * c, 16)] = inbuf[pl.ds(sh + 16 * c, 16)]
            return carry

        lax.fori_loop(0, nv16 - 1, copy_vec, 0)
        cb = nv16 - 1
        v = inbuf[pl.ds(sh + 16 * cb, 16)]
        outbuf[pl.ds(16 * cb, 16)] = jnp.where(16 * cb + lane < nv, v, neg_inf)

        def poison_vec(c, carry, outbuf=outbuf):
            outbuf[pl.ds(16 * c, 16)] = neg_inf
            return carry

        lax.fori_loop(nv16, prev16[b], poison_vec, 0)
        prev16[b] = nv16

        pltpu.async_copy(outbuf, grid_hbm.at[r0 + j], sos[b])
        if j + 2 < _ROWS_PER_W:
            a2, _, _ = params[j + 2]
            pltpu.async_copy(
                logits_hbm.at[pl.ds(a2, _INBUF)], inbuf, sis[b])

    for b in range(2):
        pltpu.make_async_copy(
            logits_hbm.at[pl.ds(0, _XPAD)], outbufs[b], sos[b]).wait()


def _lsm_body(gw_ref, g_ref, lp_ref):
    g = gw_ref[...]
    m = jnp.max(g, axis=-1, keepdims=True)
    e = jnp.exp(g - m)
    s = jnp.sum(e, axis=-1, keepdims=True)
    lp = g - (m + jnp.log(s))
    g_ref[...] = g[:, :_X]
    lp_ref[...] = lp[:, :_X]


_BR = 256


def _log_softmax(grid_wide):
    return pl.pallas_call(
        _lsm_body,
        grid=(_Y // _BR,),
        in_specs=[pl.BlockSpec((_BR, _XPAD), lambda i: (i, 0))],
        out_specs=[
            pl.BlockSpec((_BR, _X), lambda i: (i, 0)),
            pl.BlockSpec((_BR, _X), lambda i: (i, 0)),
        ],
        out_shape=[
            jax.ShapeDtypeStruct((_Y, _X), jnp.float32),
            jax.ShapeDtypeStruct((_Y, _X), jnp.float32),
        ],
    )(grid_wide)


def kernel(logits):
    logits_pad = jnp.concatenate(
        [logits, jnp.zeros((_PAD,), jnp.float32)])
    grid_wide = _sc_scatter(logits_pad)
    grid, log_probs = _log_softmax(grid_wide)
    return (grid, log_probs)
